# Initial kernel scaffold; baseline (speedup 1.0000x reference)
import functools

import jax
import jax.numpy as jnp
from jax import lax
from jax.experimental import pallas as pl
from jax.experimental.pallas import tpu as pltpu
from jax.experimental.pallas import tpu_sc as plsc

B = 16384
C = 1000
NC = 2
NS = 16
NW = NC * NS
RPW = B // NW
CHUNK = 16
NCH = RPW // CHUNK
BUFW = CHUNK * C

_mesh = plsc.VectorSubcoreMesh(core_axis_name="c", subcore_axis_name="s")


@functools.partial(
    pl.kernel,
    mesh=_mesh,
    out_type=jax.ShapeDtypeStruct((B * C,), jnp.float32),
    scratch_types=[
        pltpu.VMEM((RPW,), jnp.int32),
        pltpu.VMEM((BUFW,), jnp.float32),
        pltpu.VMEM((BUFW,), jnp.float32),
        pltpu.SemaphoreType.DMA,
        pltpu.SemaphoreType.DMA,
    ],
)
def _onehot_sc(in_hbm, out_hbm, idx_v, buf0, buf1, sem0, sem1):
    wid = lax.axis_index("s") * NC + lax.axis_index("c")
    base_row = wid * RPW

    pltpu.sync_copy(in_hbm.at[pl.ds(base_row, RPW)], idx_v)

    row_off = lax.iota(jnp.int32, 16) * C
    ones = jnp.full((16,), 1.0, jnp.float32)
    zeros = jnp.zeros((16,), jnp.float32)

    def _zero(i, carry):
        buf0[pl.ds(i * 16, 16)] = zeros
        buf1[pl.ds(i * 16, 16)] = zeros
        return carry

    lax.fori_loop(0, BUFW // 16, _zero, 0)

    bufs = (buf0, buf1)
    sems = (sem0, sem1)

    def dst(c):
        off = pl.multiple_of((base_row + c * CHUNK) * C, 8)
        return out_hbm.at[pl.ds(off, BUFW)]

    def offs(c):
        cols = idx_v[pl.ds(c * CHUNK, 16)]
        return row_off + cols

    for b in range(2):
        plsc.store_scatter(bufs[b], [offs(b)], ones)
        pltpu.async_copy(bufs[b], dst(b), sems[b])

    def _body(c2, carry):
        for b in range(2):
            c = c2 * 2 + b
            buf, sem = bufs[b], sems[b]
            pltpu.make_async_copy(buf, dst(c), sem).wait()
            plsc.store_scatter(buf, [offs(c - 2)], zeros)
            plsc.store_scatter(buf, [offs(c)], ones)
            pltpu.async_copy(buf, dst(c), sem)
        return carry

    lax.fori_loop(1, NCH // 2, _body, 0)

    for b in range(2):
        pltpu.make_async_copy(bufs[b], dst(NCH - 2 + b), sems[b]).wait()


def kernel(inputs):
    flat = _onehot_sc(inputs.astype(jnp.int32))
    return flat.reshape(B, C)


# speedup vs baseline: 1.0380x; 1.0380x over previous
import functools

import jax
import jax.numpy as jnp
from jax import lax
from jax.experimental import pallas as pl
from jax.experimental.pallas import tpu as pltpu
from jax.experimental.pallas import tpu_sc as plsc

B = 16384
C = 1000
NC = 2
NS = 16
NW = NC * NS
RPW = B // NW
CHUNK = 16
NCH = RPW // CHUNK
BUFW = CHUNK * C

_mesh = plsc.VectorSubcoreMesh(core_axis_name="c", subcore_axis_name="s")


@functools.partial(
    pl.kernel,
    mesh=_mesh,
    out_type=jax.ShapeDtypeStruct((B * C,), jnp.float32),
    compiler_params=pltpu.CompilerParams(needs_layout_passes=False),
    scratch_types=[
        pltpu.VMEM((RPW,), jnp.int32),
        pltpu.VMEM((BUFW,), jnp.float32),
        pltpu.VMEM((BUFW,), jnp.float32),
        pltpu.SemaphoreType.DMA,
        pltpu.SemaphoreType.DMA,
    ],
)
def _onehot_sc(in_hbm, out_hbm, idx_v, buf0, buf1, sem0, sem1):
    wid = lax.axis_index("s") * NC + lax.axis_index("c")
    base_row = wid * RPW

    pltpu.sync_copy(in_hbm.at[pl.ds(base_row, RPW)], idx_v)

    row_off = lax.iota(jnp.int32, 16) * C
    ones = jnp.full((16,), 1.0, jnp.float32)
    zeros = jnp.zeros((16,), jnp.float32)

    def _zero(i, carry):
        buf0[pl.ds(i * 16, 16)] = zeros
        buf1[pl.ds(i * 16, 16)] = zeros
        return carry

    lax.fori_loop(0, BUFW // 16, _zero, 0)

    bufs = (buf0, buf1)
    sems = (sem0, sem1)

    def dst(c):
        off = pl.multiple_of((base_row + c * CHUNK) * C, 8)
        return out_hbm.at[pl.ds(off, BUFW)]

    def offs(c):
        cols = idx_v[pl.ds(c * CHUNK, 16)]
        return row_off + cols

    for b in range(2):
        plsc.store_scatter(bufs[b], [offs(b)], ones)
        pltpu.async_copy(bufs[b], dst(b), sems[b])

    def _body(c2, carry):
        for b in range(2):
            c = c2 * 2 + b
            buf, sem = bufs[b], sems[b]
            pltpu.make_async_copy(buf, dst(c), sem).wait()
            plsc.store_scatter(buf, [offs(c - 2)], zeros)
            plsc.store_scatter(buf, [offs(c)], ones)
            pltpu.async_copy(buf, dst(c), sem)
        return carry

    lax.fori_loop(1, NCH // 2, _body, 0)

    for b in range(2):
        pltpu.make_async_copy(bufs[b], dst(NCH - 2 + b), sems[b]).wait()


def kernel(inputs):
    flat = _onehot_sc(inputs.astype(jnp.int32))
    return flat.reshape(B, C)


# speedup vs baseline: 1.5141x; 1.4586x over previous
import functools

import jax
import jax.numpy as jnp
from jax import lax
from jax.experimental import pallas as pl
from jax.experimental.pallas import tpu as pltpu
from jax.experimental.pallas import tpu_sc as plsc

B = 16384
C = 1000
NC = 2
NS = 16
NW = NC * NS
RPW = B // NW
CHUNK = 16
NCH = RPW // CHUNK

_mesh = plsc.VectorSubcoreMesh(core_axis_name="c", subcore_axis_name="s")


@functools.partial(
    pl.kernel,
    mesh=_mesh,
    out_type=jax.ShapeDtypeStruct((B, C), jnp.float32),
    compiler_params=pltpu.CompilerParams(needs_layout_passes=False),
    scratch_types=[
        pltpu.VMEM((RPW,), jnp.int32),
        pltpu.VMEM((CHUNK, C), jnp.float32),
        pltpu.VMEM((CHUNK, C), jnp.float32),
        pltpu.SemaphoreType.DMA,
        pltpu.SemaphoreType.DMA,
    ],
)
def _onehot_sc(in_hbm, zero_hbm, out_hbm, idx_v, buf0, buf1, sem0, sem1):
    wid = lax.axis_index("s") * NC + lax.axis_index("c")
    base_row = wid * RPW

    pltpu.sync_copy(in_hbm.at[pl.ds(base_row, RPW)], idx_v)

    rows16 = lax.iota(jnp.int32, 16)
    ones = jnp.full((16,), 1.0, jnp.float32)
    zeros = jnp.zeros((16,), jnp.float32)

    pltpu.sync_copy(zero_hbm, buf0)
    pltpu.sync_copy(zero_hbm, buf1)

    bufs = (buf0, buf1)
    sems = (sem0, sem1)

    def dst(c):
        row0 = pl.multiple_of(base_row + c * CHUNK, 8)
        return out_hbm.at[pl.ds(row0, CHUNK)]

    def cols(c):
        return idx_v[pl.ds(c * CHUNK, 16)]

    for b in range(2):
        plsc.store_scatter(bufs[b], [rows16, cols(b)], ones)
        pltpu.async_copy(bufs[b], dst(b), sems[b])

    def _body(c2, carry):
        for b in range(2):
            c = c2 * 2 + b
            buf, sem = bufs[b], sems[b]
            pltpu.make_async_copy(buf, dst(c), sem).wait()
            plsc.store_scatter(buf, [rows16, cols(c - 2)], zeros)
            plsc.store_scatter(buf, [rows16, cols(c)], ones)
            pltpu.async_copy(buf, dst(c), sem)
        return carry

    lax.fori_loop(1, NCH // 2, _body, 0)

    for b in range(2):
        pltpu.make_async_copy(bufs[b], dst(NCH - 2 + b), sems[b]).wait()


def kernel(inputs):
    zero_chunk = jnp.zeros((CHUNK, C), jnp.float32)
    return _onehot_sc(inputs.astype(jnp.int32), zero_chunk)


# speedup vs baseline: 3.1425x; 2.0755x over previous
import functools

import jax
import jax.numpy as jnp
from jax import lax
from jax.experimental import pallas as pl
from jax.experimental.pallas import tpu as pltpu
from jax.experimental.pallas import tpu_sc as plsc

B = 16384
C = 1000
NC = 2
NS = 16
NW = NC * NS
BPW = B // NW
BT = 128
NK = BPW // BT
CC = 200
NM = C // CC
NCH = NK * NM

_mesh = plsc.VectorSubcoreMesh(core_axis_name="c", subcore_axis_name="s")


@functools.partial(
    pl.kernel,
    mesh=_mesh,
    out_type=jax.ShapeDtypeStruct((C, B), jnp.float32),
    compiler_params=pltpu.CompilerParams(needs_layout_passes=False),
    scratch_types=[
        pltpu.VMEM((BPW,), jnp.int32),
        pltpu.VMEM((CC, BT), jnp.float32),
        pltpu.VMEM((CC, BT), jnp.float32),
        pltpu.SemaphoreType.DMA,
        pltpu.SemaphoreType.DMA,
    ],
)
def _onehot_sc(in_hbm, zero_hbm, out_hbm, idx_v, buf0, buf1, sem0, sem1):
    wid = lax.axis_index("s") * NC + lax.axis_index("c")
    base_col = wid * BPW

    pltpu.sync_copy(in_hbm.at[pl.ds(base_col, BPW)], idx_v)

    lanes = lax.iota(jnp.int32, 16)
    ones = jnp.full((16,), 1.0, jnp.float32)
    zeros = jnp.zeros((16,), jnp.float32)

    pltpu.sync_copy(zero_hbm, buf0)
    pltpu.sync_copy(zero_hbm, buf1)

    bufs = (buf0, buf1)
    sems = (sem0, sem1)

    def dst(ci):
        k = ci // NM
        m = ci % NM
        c0 = pl.multiple_of(m * CC, 8)
        b0 = pl.multiple_of(base_col + k * BT, 128)
        return out_hbm.at[pl.ds(c0, CC), pl.ds(b0, BT)]

    def scatter(buf, ci, val):
        k = ci // NM
        c0 = (ci % NM) * CC
        for v in range(8):
            cols = idx_v[pl.ds(k * BT + v * 16, 16)]
            local_c = cols - c0
            mask = (local_c >= 0) & (local_c < CC)
            local_c = jnp.where(mask, local_c, 0)
            local_b = lanes + (v * 16)
            plsc.store_scatter(buf, [local_c, local_b], val, mask=mask)

    for b in range(2):
        scatter(bufs[b], b, ones)
        pltpu.async_copy(bufs[b], dst(b), sems[b])

    def _body(c2, carry):
        for b in range(2):
            ci = c2 * 2 + b
            buf, sem = bufs[b], sems[b]
            pltpu.make_async_copy(buf, dst(ci), sem).wait()
            scatter(buf, ci - 2, zeros)
            scatter(buf, ci, ones)
            pltpu.async_copy(buf, dst(ci), sem)
        return carry

    lax.fori_loop(1, NCH // 2, _body, 0)

    for b in range(2):
        pltpu.make_async_copy(bufs[b], dst(NCH - 2 + b), sems[b]).wait()


def kernel(inputs):
    zero_chunk = jnp.zeros((CC, BT), jnp.float32)
    out_t = _onehot_sc(inputs.astype(jnp.int32), zero_chunk)
    return out_t.T
